# Initial kernel scaffold; baseline (speedup 1.0000x reference)
import jax
import jax.numpy as jnp
from jax.experimental import pallas as pl


def kernel(x, r, edge_index, edge_type, W, bias, alpha, bn_gamma, bn_beta):
    raise NotImplementedError("write your pallas kernel here")


# speedup vs baseline: 8.9213x; 8.9213x over previous
import functools

import jax
import jax.numpy as jnp
from jax import lax
from jax.experimental import pallas as pl
from jax.experimental.pallas import tpu as pltpu
from jax.experimental.pallas import tpu_sc as plsc

EPS = 1e-5
NC = 2
NS = 16
NW = NC * NS
L = 16
CH = 128


def _mm_body(x_ref, w_ref, o_ref):
    o_ref[...] = jnp.dot(x_ref[...], w_ref[...],
                         preferred_element_type=jnp.float32)


def _matmul(x, W, blk):
    n, d_in = x.shape
    d_out = W.shape[1]
    return pl.pallas_call(
        _mm_body,
        grid=(n // blk,),
        in_specs=[
            pl.BlockSpec((blk, d_in), lambda i: (i, 0)),
            pl.BlockSpec((d_in, d_out), lambda i: (0, 0)),
        ],
        out_specs=pl.BlockSpec((blk, d_out), lambda i: (i, 0)),
        out_shape=jax.ShapeDtypeStruct((n, d_out), jnp.float32),
    )(x, W)


def _sc_agg_body(n_nodes, nchunk, zrows, atab_len,
                 src_hbm, dst_hbm, et_hbm, alpha_hbm, xw_hbm, out_hbm,
                 src_v, dst_v, et_v, atab_v, ach_v, msg_v, acc_sh,
                 sem):
    c = lax.axis_index("c")
    s = lax.axis_index("s")
    w = s * NC + c
    rows_per_tile = n_nodes // NS

    pltpu.sync_copy(src_hbm.at[w], src_v)
    pltpu.sync_copy(dst_hbm.at[w], dst_v)
    pltpu.sync_copy(et_hbm.at[w], et_v)
    pltpu.sync_copy(alpha_hbm, atab_v)

    def _zrow(i, _):
        for c8 in range(8):
            msg_v[i, pl.ds(c8 * L, L)] = jnp.zeros((L,), jnp.float32)
        return 0
    lax.fori_loop(0, zrows, _zrow, 0)
    for m in range(rows_per_tile // zrows):
        pltpu.sync_copy(
            msg_v.at[pl.ds(0, zrows)],
            acc_sh.at[pl.ds(s * rows_per_tile + m * zrows, zrows)])
    plsc.subcore_barrier()

    def _chunk(j, _):
        for l in range(CH // L):
            et16 = et_v[j, pl.ds(l * L, L)]
            ach_v[pl.ds(l * L, L)] = plsc.load_gather(atab_v, [et16])

        def _one_dir(gidx_ref, sidx_ref):
            pltpu.async_copy(xw_hbm.at[gidx_ref], msg_v, sem).wait()

            def _row(i, _):
                a = plsc.load_gather(ach_v, [jnp.full((L,), i, jnp.int32)])
                for c8 in range(8):
                    sl = pl.ds(c8 * L, L)
                    msg_v[i, sl] = msg_v[i, sl] * a
                return 0
            lax.fori_loop(0, CH, _row, 0)

            pltpu.sync_copy(msg_v, acc_sh.at[sidx_ref], add=True)

        _one_dir(dst_v.at[j], src_v.at[j])
        _one_dir(src_v.at[j], dst_v.at[j])
        return 0
    lax.fori_loop(0, nchunk, _chunk, 0)

    plsc.subcore_barrier()
    pltpu.sync_copy(acc_sh.at[pl.ds(s * rows_per_tile, rows_per_tile)],
                    out_hbm.at[c, s])


def _sc_aggregate(src3, dst3, et3, atab, XW):
    n_nodes, d = XW.shape
    nchunk = src3.shape[1]
    rows_per_tile = n_nodes // NS
    zrows = 125
    mesh = plsc.VectorSubcoreMesh(core_axis_name="c", subcore_axis_name="s",
                                  num_cores=NC, num_subcores=NS)
    body = functools.partial(_sc_agg_body, n_nodes, nchunk, zrows,
                             atab.shape[0])
    run = pl.kernel(
        body,
        out_type=jax.ShapeDtypeStruct((NC, NS, rows_per_tile, d),
                                      jnp.float32),
        mesh=mesh,
        scratch_types=[
            pltpu.VMEM((nchunk, CH), jnp.int32),
            pltpu.VMEM((nchunk, CH), jnp.int32),
            pltpu.VMEM((nchunk, CH), jnp.int32),
            pltpu.VMEM((atab.shape[0],), jnp.float32),
            pltpu.VMEM((CH,), jnp.float32),
            pltpu.VMEM((CH, d), jnp.float32),
            pltpu.VMEM_SHARED((n_nodes, d), jnp.float32),
            pltpu.SemaphoreType.DMA,
        ],
        compiler_params=pltpu.CompilerParams(needs_layout_passes=False),
    )
    return run(src3, dst3, et3, atab, XW)


def _fin_body(n_nodes, coef_ref, pr_ref, xw_ref, bias_ref, gam_ref, bet_ref,
              o_ref, sum_scr, sq_scr):
    p = pl.program_id(0)
    j = pl.program_id(1)
    t = (pr_ref[0] + pr_ref[1] + coef_ref[0, 0] * xw_ref[...]
         + bias_ref[...])

    @pl.when(p == 0)
    def _():
        @pl.when(j == 0)
        def _():
            sum_scr[...] = jnp.zeros_like(sum_scr)
            sq_scr[...] = jnp.zeros_like(sq_scr)
        sum_scr[...] += jnp.sum(t, axis=0, keepdims=True)
        sq_scr[...] += jnp.sum(t * t, axis=0, keepdims=True)

    @pl.when(p == 1)
    def _():
        mean = sum_scr[...] / n_nodes
        var = sq_scr[...] / n_nodes - mean * mean
        o_ref[...] = ((t - mean) * lax.rsqrt(var + EPS) * gam_ref[...]
                      + bet_ref[...])


def _finale(coef, pr, XW, bias, gamma, beta, blk):
    n, d = XW.shape
    nb = n // blk
    return pl.pallas_call(
        functools.partial(_fin_body, n),
        grid=(2, nb),
        in_specs=[
            pl.BlockSpec((1, 1), lambda p, j: (0, 0)),
            pl.BlockSpec((2, blk, d), lambda p, j: (0, j, 0)),
            pl.BlockSpec((blk, d), lambda p, j: (j, 0)),
            pl.BlockSpec((1, d), lambda p, j: (0, 0)),
            pl.BlockSpec((1, d), lambda p, j: (0, 0)),
            pl.BlockSpec((1, d), lambda p, j: (0, 0)),
        ],
        out_specs=pl.BlockSpec((blk, d), lambda p, j: (j, 0)),
        out_shape=jax.ShapeDtypeStruct((n, d), jnp.float32),
        scratch_shapes=[
            pltpu.VMEM((1, d), jnp.float32),
            pltpu.VMEM((1, d), jnp.float32),
        ],
    )(coef, pr, XW, bias, gamma, beta)


def kernel(x, r, edge_index, edge_type, W, bias, alpha, bn_gamma, bn_beta):
    n_nodes, d_in = x.shape
    d_out = W.shape[1]
    n_edges = edge_index.shape[1]
    n_alpha = alpha.shape[0]
    r_last = n_alpha - 2

    XW = _matmul(x.astype(jnp.float32), W.astype(jnp.float32), blk=1000)

    nchunk = -(-n_edges // (NW * CH))
    epad = NW * nchunk * CH
    src = edge_index[0].astype(jnp.int32)
    dst = edge_index[1].astype(jnp.int32)
    et = edge_type.astype(jnp.int32)
    zpad = jnp.zeros((epad - n_edges,), jnp.int32)
    src3 = jnp.concatenate([src, zpad]).reshape(NW, nchunk, CH)
    dst3 = jnp.concatenate([dst, zpad]).reshape(NW, nchunk, CH)
    et3 = jnp.concatenate([et, zpad]).reshape(NW, nchunk, CH)

    atab_len = -(-n_alpha // L) * L
    atab = jnp.pad(alpha[:, 0].astype(jnp.float32),
                   (0, atab_len - n_alpha))

    partial = _sc_aggregate(src3, dst3, et3, atab, XW)
    pr = partial.reshape(NC, n_nodes, d_out)

    coef = (2.0 * alpha[r_last]).astype(jnp.float32).reshape(1, 1)
    out = _finale(coef, pr, XW,
                  bias.astype(jnp.float32).reshape(1, d_out),
                  bn_gamma.astype(jnp.float32).reshape(1, d_out),
                  bn_beta.astype(jnp.float32).reshape(1, d_out),
                  blk=1000)
    return (out, r)


# speedup vs baseline: 9.3725x; 1.0506x over previous
import functools

import jax
import jax.numpy as jnp
from jax import lax
from jax.experimental import pallas as pl
from jax.experimental.pallas import tpu as pltpu
from jax.experimental.pallas import tpu_sc as plsc

EPS = 1e-5
NC = 2
NS = 16
NW = NC * NS
L = 16
CH = 128


def _mm_body(x_ref, w_ref, o_ref):
    o_ref[...] = jnp.dot(x_ref[...], w_ref[...],
                         preferred_element_type=jnp.float32)


def _matmul(x, W, blk):
    n, d_in = x.shape
    d_out = W.shape[1]
    return pl.pallas_call(
        _mm_body,
        grid=(n // blk,),
        in_specs=[
            pl.BlockSpec((blk, d_in), lambda i: (i, 0)),
            pl.BlockSpec((d_in, d_out), lambda i: (0, 0)),
        ],
        out_specs=pl.BlockSpec((blk, d_out), lambda i: (i, 0)),
        out_shape=jax.ShapeDtypeStruct((n, d_out), jnp.float32),
    )(x, W)


def _sc_agg_body(n_nodes, nchunk, zrows,
                 g4_hbm, s4_hbm, et3_hbm, alpha_hbm, xw_hbm, out_hbm,
                 gidx, sidx, etb, ach, msg, atab_v, acc_sh, semi, semg):
    c = lax.axis_index("c")
    s = lax.axis_index("s")
    w = s * NC + c
    rows_per_tile = n_nodes // NS
    n_units = 2 * nchunk

    pltpu.sync_copy(alpha_hbm, atab_v)

    def _zrow(i, _):
        for c8 in range(8):
            msg[0][i, pl.ds(c8 * L, L)] = jnp.zeros((L,), jnp.float32)
        return 0
    lax.fori_loop(0, zrows, _zrow, 0)
    for m in range(rows_per_tile // zrows):
        pltpu.sync_copy(
            msg[0].at[pl.ds(0, zrows)],
            acc_sh.at[pl.ds(s * rows_per_tile + m * zrows, zrows)])
    plsc.subcore_barrier()

    def _load_idx(u_j, u_d, p):
        pltpu.async_copy(g4_hbm.at[w, u_j, u_d], gidx[p], semi[p])
        pltpu.async_copy(s4_hbm.at[w, u_j, u_d], sidx[p], semi[p])
        pltpu.async_copy(et3_hbm.at[w, u_j], etb[p], semi[p])

    def _wait_idx(u_j, u_d, p):
        pltpu.make_async_copy(g4_hbm.at[w, u_j, u_d], gidx[p], semi[p]).wait()
        pltpu.make_async_copy(s4_hbm.at[w, u_j, u_d], sidx[p], semi[p]).wait()
        pltpu.make_async_copy(et3_hbm.at[w, u_j], etb[p], semi[p]).wait()

    def _compute_ach(p):
        for l in range(CH // L):
            et16 = etb[p][pl.ds(l * L, L)]
            ach[p][pl.ds(l * L, L)] = plsc.load_gather(atab_v, [et16])

    def _unit(u, p_cur):
        p_nxt = 1 - p_cur

        @pl.when(u + 1 < n_units)
        def _():
            nj, nd = (u + 1) // 2, (u + 1) % 2
            _wait_idx(nj, nd, p_nxt)
            _compute_ach(p_nxt)
            pltpu.async_copy(xw_hbm.at[gidx[p_nxt]], msg[p_nxt],
                             semg[p_nxt])

        pltpu.make_async_copy(xw_hbm.at[gidx[p_cur]], msg[p_cur],
                              semg[p_cur]).wait()

        def _row(i, _):
            a = plsc.load_gather(ach[p_cur], [jnp.full((L,), i, jnp.int32)])
            for c8 in range(8):
                sl = pl.ds(c8 * L, L)
                msg[p_cur][i, sl] = msg[p_cur][i, sl] * a
            return 0
        lax.fori_loop(0, CH, _row, 0)

        pltpu.sync_copy(msg[p_cur], acc_sh.at[sidx[p_cur]], add=True)

        @pl.when(u + 2 < n_units)
        def _():
            nj, nd = (u + 2) // 2, (u + 2) % 2
            _load_idx(nj, nd, p_cur)

    _load_idx(0, 0, 0)
    _wait_idx(0, 0, 0)
    _compute_ach(0)
    pltpu.async_copy(xw_hbm.at[gidx[0]], msg[0], semg[0])
    _load_idx(0, 1, 1)

    def _pair(i, _):
        _unit(2 * i, 0)
        _unit(2 * i + 1, 1)
        return 0
    lax.fori_loop(0, n_units // 2, _pair, 0)

    plsc.subcore_barrier()
    pltpu.sync_copy(acc_sh.at[pl.ds(s * rows_per_tile, rows_per_tile)],
                    out_hbm.at[c, s])


def _sc_aggregate(g4, s4, et3, atab, XW):
    n_nodes, d = XW.shape
    nchunk = g4.shape[1]
    rows_per_tile = n_nodes // NS
    zrows = 125
    mesh = plsc.VectorSubcoreMesh(core_axis_name="c", subcore_axis_name="s",
                                  num_cores=NC, num_subcores=NS)
    body = functools.partial(_sc_agg_body, n_nodes, nchunk, zrows)
    run = pl.kernel(
        body,
        out_type=jax.ShapeDtypeStruct((NC, NS, rows_per_tile, d),
                                      jnp.float32),
        mesh=mesh,
        scratch_types=[
            [pltpu.VMEM((CH,), jnp.int32)] * 2,
            [pltpu.VMEM((CH,), jnp.int32)] * 2,
            [pltpu.VMEM((CH,), jnp.int32)] * 2,
            [pltpu.VMEM((CH,), jnp.float32)] * 2,
            [pltpu.VMEM((CH, d), jnp.float32)] * 2,
            pltpu.VMEM((atab.shape[0],), jnp.float32),
            pltpu.VMEM_SHARED((n_nodes, d), jnp.float32),
            [pltpu.SemaphoreType.DMA] * 2,
            [pltpu.SemaphoreType.DMA] * 2,
        ],
        compiler_params=pltpu.CompilerParams(needs_layout_passes=False),
    )
    return run(g4, s4, et3, atab, XW)


def _fin_body(n_nodes, coef_ref, pr_ref, xw_ref, bias_ref, gam_ref, bet_ref,
              o_ref, sum_scr, sq_scr):
    p = pl.program_id(0)
    j = pl.program_id(1)
    t = (pr_ref[0] + pr_ref[1] + coef_ref[0, 0] * xw_ref[...]
         + bias_ref[...])

    @pl.when(p == 0)
    def _():
        @pl.when(j == 0)
        def _():
            sum_scr[...] = jnp.zeros_like(sum_scr)
            sq_scr[...] = jnp.zeros_like(sq_scr)
        sum_scr[...] += jnp.sum(t, axis=0, keepdims=True)
        sq_scr[...] += jnp.sum(t * t, axis=0, keepdims=True)

    @pl.when(p == 1)
    def _():
        mean = sum_scr[...] / n_nodes
        var = sq_scr[...] / n_nodes - mean * mean
        o_ref[...] = ((t - mean) * lax.rsqrt(var + EPS) * gam_ref[...]
                      + bet_ref[...])


def _finale(coef, pr, XW, bias, gamma, beta, blk):
    n, d = XW.shape
    nb = n // blk
    return pl.pallas_call(
        functools.partial(_fin_body, n),
        grid=(2, nb),
        in_specs=[
            pl.BlockSpec((1, 1), lambda p, j: (0, 0)),
            pl.BlockSpec((2, blk, d), lambda p, j: (0, j, 0)),
            pl.BlockSpec((blk, d), lambda p, j: (j, 0)),
            pl.BlockSpec((1, d), lambda p, j: (0, 0)),
            pl.BlockSpec((1, d), lambda p, j: (0, 0)),
            pl.BlockSpec((1, d), lambda p, j: (0, 0)),
        ],
        out_specs=pl.BlockSpec((blk, d), lambda p, j: (j, 0)),
        out_shape=jax.ShapeDtypeStruct((n, d), jnp.float32),
        scratch_shapes=[
            pltpu.VMEM((1, d), jnp.float32),
            pltpu.VMEM((1, d), jnp.float32),
        ],
    )(coef, pr, XW, bias, gamma, beta)


def kernel(x, r, edge_index, edge_type, W, bias, alpha, bn_gamma, bn_beta):
    n_nodes, d_in = x.shape
    d_out = W.shape[1]
    n_edges = edge_index.shape[1]
    n_alpha = alpha.shape[0]
    r_last = n_alpha - 2

    XW = _matmul(x.astype(jnp.float32), W.astype(jnp.float32), blk=1000)

    nchunk = -(-n_edges // (NW * CH))
    epad = NW * nchunk * CH
    src = edge_index[0].astype(jnp.int32)
    dst = edge_index[1].astype(jnp.int32)
    et = edge_type.astype(jnp.int32)
    zpad = jnp.zeros((epad - n_edges,), jnp.int32)
    src3 = jnp.concatenate([src, zpad]).reshape(NW, nchunk, CH)
    dst3 = jnp.concatenate([dst, zpad]).reshape(NW, nchunk, CH)
    et3 = jnp.concatenate([et, zpad]).reshape(NW, nchunk, CH)
    g4 = jnp.stack([dst3, src3], axis=2)
    s4 = jnp.stack([src3, dst3], axis=2)

    atab_len = -(-n_alpha // L) * L
    atab = jnp.pad(alpha[:, 0].astype(jnp.float32),
                   (0, atab_len - n_alpha))

    partial = _sc_aggregate(g4, s4, et3, atab, XW)
    pr = partial.reshape(NC, n_nodes, d_out)

    coef = (2.0 * alpha[r_last]).astype(jnp.float32).reshape(1, 1)
    out = _finale(coef, pr, XW,
                  bias.astype(jnp.float32).reshape(1, d_out),
                  bn_gamma.astype(jnp.float32).reshape(1, d_out),
                  bn_beta.astype(jnp.float32).reshape(1, d_out),
                  blk=1000)
    return (out, r)


# speedup vs baseline: 13.7980x; 1.4722x over previous
import functools

import jax
import jax.numpy as jnp
from jax import lax
from jax.experimental import pallas as pl
from jax.experimental.pallas import tpu as pltpu
from jax.experimental.pallas import tpu_sc as plsc

EPS = 1e-5
NC = 2
NS = 16
NW = NC * NS
L = 16
CH = 96
NB = 3
NQ = 6


def _mm_body(x_ref, w_ref, o_ref):
    o_ref[...] = jnp.dot(x_ref[...], w_ref[...],
                         preferred_element_type=jnp.float32)


def _matmul(x, W, blk):
    n, d_in = x.shape
    d_out = W.shape[1]
    return pl.pallas_call(
        _mm_body,
        grid=(n // blk,),
        in_specs=[
            pl.BlockSpec((blk, d_in), lambda i: (i, 0)),
            pl.BlockSpec((d_in, d_out), lambda i: (0, 0)),
        ],
        out_specs=pl.BlockSpec((blk, d_out), lambda i: (i, 0)),
        out_shape=jax.ShapeDtypeStruct((n, d_out), jnp.float32),
    )(x, W)


def _sc_agg_body(n_nodes, nchunk, zrows,
                 g4_hbm, s4_hbm, et3_hbm, alpha_hbm, xw_hbm, out_hbm,
                 gidx, sidx, etb, ach, msg, atab_v, acc_sh,
                 semi, semg, sems):
    c = lax.axis_index("c")
    s = lax.axis_index("s")
    w = s * NC + c
    rows_per_tile = n_nodes // NS
    n_units = 2 * nchunk

    pltpu.sync_copy(alpha_hbm, atab_v)

    def _zrow(i, _):
        for c8 in range(8):
            msg[0][i, pl.ds(c8 * L, L)] = jnp.zeros((L,), jnp.float32)
        return 0
    lax.fori_loop(0, zrows, _zrow, 0)
    for m in range(rows_per_tile // zrows):
        pltpu.sync_copy(
            msg[0].at[pl.ds(0, zrows)],
            acc_sh.at[pl.ds(s * rows_per_tile + m * zrows, zrows)])
    plsc.subcore_barrier()

    def _load_idx(u_j, u_d, q):
        pltpu.async_copy(g4_hbm.at[w, u_j, u_d], gidx[q], semi[q])
        pltpu.async_copy(s4_hbm.at[w, u_j, u_d], sidx[q], semi[q])
        pltpu.async_copy(et3_hbm.at[w, u_j], etb[q], semi[q])

    def _wait_idx(u_j, u_d, q):
        pltpu.make_async_copy(g4_hbm.at[w, u_j, u_d], gidx[q], semi[q]).wait()
        pltpu.make_async_copy(s4_hbm.at[w, u_j, u_d], sidx[q], semi[q]).wait()
        pltpu.make_async_copy(et3_hbm.at[w, u_j], etb[q], semi[q]).wait()

    def _compute_ach(q):
        for l in range(CH // L):
            et16 = etb[q][pl.ds(l * L, L)]
            ach[q][pl.ds(l * L, L)] = plsc.load_gather(atab_v, [et16])

    def _wait_scat(p, q):
        pltpu.make_async_copy(msg[p], acc_sh.at[sidx[q]], sems[p]).wait()

    def _unit(u, r):
        p, q = r % NB, r % NQ
        pn, qn = (r + 1) % NB, (r + 1) % NQ

        @pl.when(u + 1 < n_units)
        def _():
            nj, nd = (u + 1) // 2, (u + 1) % 2
            _wait_idx(nj, nd, qn)
            _compute_ach(qn)

            @pl.when(u >= 2)
            def _():
                _wait_scat(pn, (r + 4) % NQ)
            pltpu.async_copy(xw_hbm.at[gidx[qn]], msg[pn], semg[pn])

        pltpu.make_async_copy(xw_hbm.at[gidx[q]], msg[p], semg[p]).wait()

        @plsc.parallel_loop(0, CH, unroll=2)
        def _row(i):
            a = plsc.load_gather(ach[q], [jnp.full((L,), i, jnp.int32)])
            for c8 in range(8):
                sl = pl.ds(c8 * L, L)
                msg[p][i, sl] = msg[p][i, sl] * a

        pltpu.async_copy(msg[p], acc_sh.at[sidx[q]], sems[p], add=True)

        @pl.when(u + 2 < n_units)
        def _():
            nj, nd = (u + 2) // 2, (u + 2) % 2
            _load_idx(nj, nd, (r + 2) % NQ)

    _load_idx(0, 0, 0)
    _load_idx(0, 1, 1)
    _wait_idx(0, 0, 0)
    _compute_ach(0)
    pltpu.async_copy(xw_hbm.at[gidx[0]], msg[0], semg[0])

    def _six(i, _):
        for r in range(NQ):
            _unit(NQ * i + r, r)
        return 0
    lax.fori_loop(0, n_units // NQ, _six, 0)

    _wait_scat((n_units - 2) % NB, (n_units - 2) % NQ)
    _wait_scat((n_units - 1) % NB, (n_units - 1) % NQ)

    plsc.subcore_barrier()
    pltpu.sync_copy(acc_sh.at[pl.ds(s * rows_per_tile, rows_per_tile)],
                    out_hbm.at[c, s])


def _sc_aggregate(g4, s4, et3, atab, XW):
    n_nodes, d = XW.shape
    nchunk = g4.shape[1]
    rows_per_tile = n_nodes // NS
    zrows = 25
    mesh = plsc.VectorSubcoreMesh(core_axis_name="c", subcore_axis_name="s",
                                  num_cores=NC, num_subcores=NS)
    body = functools.partial(_sc_agg_body, n_nodes, nchunk, zrows)
    run = pl.kernel(
        body,
        out_type=jax.ShapeDtypeStruct((NC, NS, rows_per_tile, d),
                                      jnp.float32),
        mesh=mesh,
        scratch_types=[
            [pltpu.VMEM((CH,), jnp.int32)] * NQ,
            [pltpu.VMEM((CH,), jnp.int32)] * NQ,
            [pltpu.VMEM((CH,), jnp.int32)] * NQ,
            [pltpu.VMEM((CH,), jnp.float32)] * NQ,
            [pltpu.VMEM((CH, d), jnp.float32)] * NB,
            pltpu.VMEM((atab.shape[0],), jnp.float32),
            pltpu.VMEM_SHARED((n_nodes, d), jnp.float32),
            [pltpu.SemaphoreType.DMA] * NQ,
            [pltpu.SemaphoreType.DMA] * NB,
            [pltpu.SemaphoreType.DMA] * NB,
        ],
        compiler_params=pltpu.CompilerParams(needs_layout_passes=False),
    )
    return run(g4, s4, et3, atab, XW)


def _fin_body(n_nodes, coef_ref, pr_ref, xw_ref, bias_ref, gam_ref, bet_ref,
              o_ref, sum_scr, sq_scr):
    p = pl.program_id(0)
    j = pl.program_id(1)
    t = (pr_ref[0] + pr_ref[1] + coef_ref[0, 0] * xw_ref[...]
         + bias_ref[...])

    @pl.when(p == 0)
    def _():
        @pl.when(j == 0)
        def _():
            sum_scr[...] = jnp.zeros_like(sum_scr)
            sq_scr[...] = jnp.zeros_like(sq_scr)
        sum_scr[...] += jnp.sum(t, axis=0, keepdims=True)
        sq_scr[...] += jnp.sum(t * t, axis=0, keepdims=True)

    @pl.when(p == 1)
    def _():
        mean = sum_scr[...] / n_nodes
        var = sq_scr[...] / n_nodes - mean * mean
        o_ref[...] = ((t - mean) * lax.rsqrt(var + EPS) * gam_ref[...]
                      + bet_ref[...])


def _finale(coef, pr, XW, bias, gamma, beta, blk):
    n, d = XW.shape
    nb = n // blk
    return pl.pallas_call(
        functools.partial(_fin_body, n),
        grid=(2, nb),
        in_specs=[
            pl.BlockSpec((1, 1), lambda p, j: (0, 0)),
            pl.BlockSpec((2, blk, d), lambda p, j: (0, j, 0)),
            pl.BlockSpec((blk, d), lambda p, j: (j, 0)),
            pl.BlockSpec((1, d), lambda p, j: (0, 0)),
            pl.BlockSpec((1, d), lambda p, j: (0, 0)),
            pl.BlockSpec((1, d), lambda p, j: (0, 0)),
        ],
        out_specs=pl.BlockSpec((blk, d), lambda p, j: (j, 0)),
        out_shape=jax.ShapeDtypeStruct((n, d), jnp.float32),
        scratch_shapes=[
            pltpu.VMEM((1, d), jnp.float32),
            pltpu.VMEM((1, d), jnp.float32),
        ],
    )(coef, pr, XW, bias, gamma, beta)


def kernel(x, r, edge_index, edge_type, W, bias, alpha, bn_gamma, bn_beta):
    n_nodes, d_in = x.shape
    d_out = W.shape[1]
    n_edges = edge_index.shape[1]
    n_alpha = alpha.shape[0]
    r_last = n_alpha - 2

    XW = _matmul(x.astype(jnp.float32), W.astype(jnp.float32), blk=1000)

    nchunk = 3 * (-(-n_edges // (NW * CH * 3)))
    epad = NW * nchunk * CH
    src = edge_index[0].astype(jnp.int32)
    dst = edge_index[1].astype(jnp.int32)
    et = edge_type.astype(jnp.int32)
    zpad = jnp.zeros((epad - n_edges,), jnp.int32)
    src3 = jnp.concatenate([src, zpad]).reshape(NW, nchunk, CH)
    dst3 = jnp.concatenate([dst, zpad]).reshape(NW, nchunk, CH)
    et3 = jnp.concatenate([et, zpad]).reshape(NW, nchunk, CH)
    g4 = jnp.stack([dst3, src3], axis=2)
    s4 = jnp.stack([src3, dst3], axis=2)

    atab_len = -(-n_alpha // L) * L
    atab = jnp.pad(alpha[:, 0].astype(jnp.float32),
                   (0, atab_len - n_alpha))

    partial = _sc_aggregate(g4, s4, et3, atab, XW)
    pr = partial.reshape(NC, n_nodes, d_out)

    coef = (2.0 * alpha[r_last]).astype(jnp.float32).reshape(1, 1)
    out = _finale(coef, pr, XW,
                  bias.astype(jnp.float32).reshape(1, d_out),
                  bn_gamma.astype(jnp.float32).reshape(1, d_out),
                  bn_beta.astype(jnp.float32).reshape(1, d_out),
                  blk=1000)
    return (out, r)


# speedup vs baseline: 21.6097x; 1.5661x over previous
import functools

import jax
import jax.numpy as jnp
from jax import lax
from jax.experimental import pallas as pl
from jax.experimental.pallas import tpu as pltpu
from jax.experimental.pallas import tpu_sc as plsc

EPS = 1e-5
NC = 2
NS = 16
NW = NC * NS
L = 16
CH = 96
NB = 3
NQ = 6


def _mm_body(x_ref, w_ref, o_ref):
    o_ref[...] = jnp.dot(x_ref[...], w_ref[...],
                         preferred_element_type=jnp.float32)


def _matmul(x, W, blk):
    n, d_in = x.shape
    d_out = W.shape[1]
    return pl.pallas_call(
        _mm_body,
        grid=(n // blk,),
        in_specs=[
            pl.BlockSpec((blk, d_in), lambda i: (i, 0)),
            pl.BlockSpec((d_in, d_out), lambda i: (0, 0)),
        ],
        out_specs=pl.BlockSpec((blk, d_out), lambda i: (i, 0)),
        out_shape=jax.ShapeDtypeStruct((n, d_out), jnp.float32),
    )(x, W)


def _sc_agg_body(n_nodes, nchunk, zrows,
                 g4_hbm, s4_hbm, et3_hbm, alpha_hbm, xw_hbm, out_hbm,
                 gidx, sidx, etb, ach, msg, atab_v, acc_sh,
                 semi, semg, sems):
    c = lax.axis_index("c")
    s = lax.axis_index("s")
    w = s * NC + c
    rows_per_tile = n_nodes // NS
    n_units = 2 * nchunk

    pltpu.sync_copy(alpha_hbm, atab_v)

    def _zrow(i, _):
        for c8 in range(8):
            msg[0][i, pl.ds(c8 * L, L)] = jnp.zeros((L,), jnp.float32)
        return 0
    lax.fori_loop(0, zrows, _zrow, 0)
    for m in range(rows_per_tile // zrows):
        pltpu.sync_copy(
            msg[0].at[pl.ds(0, zrows)],
            acc_sh.at[pl.ds(s * rows_per_tile + m * zrows, zrows)])
    plsc.subcore_barrier()

    def _load_idx(u_j, u_d, q):
        pltpu.async_copy(g4_hbm.at[w, u_j, u_d], gidx[q], semi[q])
        pltpu.async_copy(s4_hbm.at[w, u_j, u_d], sidx[q], semi[q])
        pltpu.async_copy(et3_hbm.at[w, u_j], etb[q], semi[q])

    def _wait_idx(u_j, u_d, q):
        pltpu.make_async_copy(g4_hbm.at[w, u_j, u_d], gidx[q], semi[q]).wait()
        pltpu.make_async_copy(s4_hbm.at[w, u_j, u_d], sidx[q], semi[q]).wait()
        pltpu.make_async_copy(et3_hbm.at[w, u_j], etb[q], semi[q]).wait()

    def _compute_ach(q):
        for l in range(CH // L):
            et16 = etb[q][pl.ds(l * L, L)]
            ach[q][pl.ds(l * L, L)] = plsc.load_gather(atab_v, [et16])

    def _wait_scat(p, q):
        pltpu.make_async_copy(msg[p], acc_sh.at[sidx[q]], sems[p]).wait()

    def _unit(u, r):
        p, q = r % NB, r % NQ
        pn, qn = (r + 1) % NB, (r + 1) % NQ

        @pl.when(u + 1 < n_units)
        def _():
            nj, nd = (u + 1) // 2, (u + 1) % 2
            _wait_idx(nj, nd, qn)
            _compute_ach(qn)

            @pl.when(u >= 2)
            def _():
                _wait_scat(pn, (r + 4) % NQ)
            pltpu.async_copy(xw_hbm.at[gidx[qn]], msg[pn], semg[pn])

        pltpu.make_async_copy(xw_hbm.at[gidx[q]], msg[p], semg[p]).wait()

        @plsc.parallel_loop(0, CH, unroll=2)
        def _row(i):
            a = plsc.load_gather(ach[q], [jnp.full((L,), i, jnp.int32)])
            for c8 in range(8):
                sl = pl.ds(c8 * L, L)
                msg[p][i, sl] = msg[p][i, sl] * a

        pltpu.async_copy(msg[p], acc_sh.at[sidx[q]], sems[p], add=True)

        @pl.when(u + 2 < n_units)
        def _():
            nj, nd = (u + 2) // 2, (u + 2) % 2
            _load_idx(nj, nd, (r + 2) % NQ)

    _load_idx(0, 0, 0)
    _load_idx(0, 1, 1)
    _wait_idx(0, 0, 0)
    _compute_ach(0)
    pltpu.async_copy(xw_hbm.at[gidx[0]], msg[0], semg[0])

    def _six(i, _):
        for r in range(NQ):
            _unit(NQ * i + r, r)
        return 0
    lax.fori_loop(0, n_units // NQ, _six, 0)

    _wait_scat((n_units - 2) % NB, (n_units - 2) % NQ)
    _wait_scat((n_units - 1) % NB, (n_units - 1) % NQ)

    plsc.subcore_barrier()
    pltpu.sync_copy(acc_sh.at[pl.ds(s * rows_per_tile, rows_per_tile)],
                    out_hbm.at[c, s])


def _sc_aggregate(g4, s4, et3, atab, XW):
    n_nodes, d = XW.shape
    nchunk = g4.shape[1]
    rows_per_tile = n_nodes // NS
    zrows = 25
    mesh = plsc.VectorSubcoreMesh(core_axis_name="c", subcore_axis_name="s",
                                  num_cores=NC, num_subcores=NS)
    body = functools.partial(_sc_agg_body, n_nodes, nchunk, zrows)
    run = pl.kernel(
        body,
        out_type=jax.ShapeDtypeStruct((NC, NS, rows_per_tile, d),
                                      jnp.float32),
        mesh=mesh,
        scratch_types=[
            [pltpu.VMEM((CH,), jnp.int32)] * NQ,
            [pltpu.VMEM((CH,), jnp.int32)] * NQ,
            [pltpu.VMEM((CH,), jnp.int32)] * NQ,
            [pltpu.VMEM((CH,), jnp.float32)] * NQ,
            [pltpu.VMEM((CH, d), jnp.float32)] * NB,
            pltpu.VMEM((atab.shape[0],), jnp.float32),
            pltpu.VMEM_SHARED((n_nodes, d), jnp.float32),
            [pltpu.SemaphoreType.DMA] * NQ,
            [pltpu.SemaphoreType.DMA] * NB,
            [pltpu.SemaphoreType.DMA] * NB,
        ],
        compiler_params=pltpu.CompilerParams(needs_layout_passes=False),
    )
    return run(g4, s4, et3, atab, XW)


def _fin_body(n_nodes, coef_ref, pr_ref, xw_ref, bias_ref, gam_ref, bet_ref,
              o_ref, sum_scr, sq_scr):
    p = pl.program_id(0)
    j = pl.program_id(1)
    t = (pr_ref[0] + pr_ref[1] + coef_ref[0, 0] * xw_ref[...]
         + bias_ref[...])

    @pl.when(p == 0)
    def _():
        @pl.when(j == 0)
        def _():
            sum_scr[...] = jnp.zeros_like(sum_scr)
            sq_scr[...] = jnp.zeros_like(sq_scr)
        sum_scr[...] += jnp.sum(t, axis=0, keepdims=True)
        sq_scr[...] += jnp.sum(t * t, axis=0, keepdims=True)

    @pl.when(p == 1)
    def _():
        mean = sum_scr[...] / n_nodes
        var = sq_scr[...] / n_nodes - mean * mean
        o_ref[...] = ((t - mean) * lax.rsqrt(var + EPS) * gam_ref[...]
                      + bet_ref[...])


def _finale(coef, pr, XW, bias, gamma, beta, blk):
    n, d = XW.shape
    nb = n // blk
    return pl.pallas_call(
        functools.partial(_fin_body, n),
        grid=(2, nb),
        in_specs=[
            pl.BlockSpec((1, 1), lambda p, j: (0, 0)),
            pl.BlockSpec((2, blk, d), lambda p, j: (0, j, 0)),
            pl.BlockSpec((blk, d), lambda p, j: (j, 0)),
            pl.BlockSpec((1, d), lambda p, j: (0, 0)),
            pl.BlockSpec((1, d), lambda p, j: (0, 0)),
            pl.BlockSpec((1, d), lambda p, j: (0, 0)),
        ],
        out_specs=pl.BlockSpec((blk, d), lambda p, j: (j, 0)),
        out_shape=jax.ShapeDtypeStruct((n, d), jnp.float32),
        scratch_shapes=[
            pltpu.VMEM((1, d), jnp.float32),
            pltpu.VMEM((1, d), jnp.float32),
        ],
    )(coef, pr, XW, bias, gamma, beta)


def kernel(x, r, edge_index, edge_type, W, bias, alpha, bn_gamma, bn_beta):
    n_nodes, d_in = x.shape
    d_out = W.shape[1]
    n_edges = edge_index.shape[1]
    n_alpha = alpha.shape[0]
    r_last = n_alpha - 2

    XW = _matmul(x.astype(jnp.float32), W.astype(jnp.float32), blk=1000)

    nchunk = 3 * (-(-n_edges // (NW * CH * 3)))
    epad = NW * nchunk * CH
    src = edge_index[0].astype(jnp.int32)
    dst = edge_index[1].astype(jnp.int32)
    et = edge_type.astype(jnp.int32)
    zpad = jnp.zeros((epad - n_edges,), jnp.int32)
    npad = (jnp.arange(epad - n_edges, dtype=jnp.int32) % n_nodes)
    src3 = jnp.concatenate([src, npad]).reshape(NW, nchunk, CH)
    dst3 = jnp.concatenate([dst, npad]).reshape(NW, nchunk, CH)
    et3 = jnp.concatenate([et, zpad]).reshape(NW, nchunk, CH)
    g4 = jnp.stack([dst3, src3], axis=2)
    s4 = jnp.stack([src3, dst3], axis=2)

    atab_len = -(-n_alpha // L) * L
    atab = jnp.pad(alpha[:, 0].astype(jnp.float32),
                   (0, atab_len - n_alpha))

    partial = _sc_aggregate(g4, s4, et3, atab, XW)
    pr = partial.reshape(NC, n_nodes, d_out)

    coef = (2.0 * alpha[r_last]).astype(jnp.float32).reshape(1, 1)
    out = _finale(coef, pr, XW,
                  bias.astype(jnp.float32).reshape(1, d_out),
                  bn_gamma.astype(jnp.float32).reshape(1, d_out),
                  bn_beta.astype(jnp.float32).reshape(1, d_out),
                  blk=1000)
    return (out, r)


# speedup vs baseline: 22.3134x; 1.0326x over previous
import functools

import jax
import jax.numpy as jnp
from jax import lax
from jax.experimental import pallas as pl
from jax.experimental.pallas import tpu as pltpu
from jax.experimental.pallas import tpu_sc as plsc

EPS = 1e-5
NC = 2
NS = 16
NW = NC * NS
L = 16
CH = 96
NB = 3
NQ = 6


def _mm_body(x_ref, w_ref, o_ref):
    o_ref[...] = jnp.dot(x_ref[...], w_ref[...],
                         preferred_element_type=jnp.float32)


def _matmul(x, W, blk):
    n, d_in = x.shape
    d_out = W.shape[1]
    return pl.pallas_call(
        _mm_body,
        grid=(n // blk,),
        in_specs=[
            pl.BlockSpec((blk, d_in), lambda i: (i, 0)),
            pl.BlockSpec((d_in, d_out), lambda i: (0, 0)),
        ],
        out_specs=pl.BlockSpec((blk, d_out), lambda i: (i, 0)),
        out_shape=jax.ShapeDtypeStruct((n, d_out), jnp.float32),
    )(x, W)


def _sc_agg_body(n_nodes, nchunk, zrows,
                 src3_hbm, dst3_hbm, et3_hbm, alpha_hbm, xw_hbm, out_hbm,
                 gidx, sidx, etb, ach, msg, atab_v, acc_sh,
                 semi, semg, sems):
    c = lax.axis_index("c")
    s = lax.axis_index("s")
    w = s * NC + c
    rows_per_tile = n_nodes // NS
    n_units = 2 * nchunk

    pltpu.sync_copy(alpha_hbm, atab_v)

    def _zrow(i, _):
        for c8 in range(8):
            msg[0][i, pl.ds(c8 * L, L)] = jnp.zeros((L,), jnp.float32)
        return 0
    lax.fori_loop(0, zrows, _zrow, 0)
    for m in range(rows_per_tile // zrows):
        pltpu.sync_copy(
            msg[0].at[pl.ds(0, zrows)],
            acc_sh.at[pl.ds(s * rows_per_tile + m * zrows, zrows)])
    plsc.subcore_barrier()

    def _load_idx(u_j, u_d, q):
        g_hbm, s_hbm = (dst3_hbm, src3_hbm) if u_d == 0 else (src3_hbm,
                                                              dst3_hbm)
        pltpu.async_copy(g_hbm.at[w, u_j], gidx[q], semi[q])
        pltpu.async_copy(s_hbm.at[w, u_j], sidx[q], semi[q])
        pltpu.async_copy(et3_hbm.at[w, u_j], etb[q], semi[q])

    def _wait_idx(u_j, u_d, q):
        g_hbm, s_hbm = (dst3_hbm, src3_hbm) if u_d == 0 else (src3_hbm,
                                                              dst3_hbm)
        pltpu.make_async_copy(g_hbm.at[w, u_j], gidx[q], semi[q]).wait()
        pltpu.make_async_copy(s_hbm.at[w, u_j], sidx[q], semi[q]).wait()
        pltpu.make_async_copy(et3_hbm.at[w, u_j], etb[q], semi[q]).wait()

    def _compute_ach(q):
        for l in range(CH // L):
            et16 = etb[q][pl.ds(l * L, L)]
            ach[q][pl.ds(l * L, L)] = plsc.load_gather(atab_v, [et16])

    def _wait_scat(p, q):
        pltpu.make_async_copy(msg[p], acc_sh.at[sidx[q]], sems[p]).wait()

    def _unit(u, r):
        p, q = r % NB, r % NQ
        pn, qn = (r + 1) % NB, (r + 1) % NQ

        @pl.when(u + 1 < n_units)
        def _():
            nj, nd = (u + 1) // 2, (r + 1) % 2
            _wait_idx(nj, nd, qn)
            _compute_ach(qn)

            @pl.when(u >= 2)
            def _():
                _wait_scat(pn, (r + 4) % NQ)
            pltpu.async_copy(xw_hbm.at[gidx[qn]], msg[pn], semg[pn])

        pltpu.make_async_copy(xw_hbm.at[gidx[q]], msg[p], semg[p]).wait()

        @plsc.parallel_loop(0, CH, unroll=2)
        def _row(i):
            a = plsc.load_gather(ach[q], [jnp.full((L,), i, jnp.int32)])
            for c8 in range(8):
                sl = pl.ds(c8 * L, L)
                msg[p][i, sl] = msg[p][i, sl] * a

        pltpu.async_copy(msg[p], acc_sh.at[sidx[q]], sems[p], add=True)

        @pl.when(u + 2 < n_units)
        def _():
            nj, nd = (u + 2) // 2, r % 2
            _load_idx(nj, nd, (r + 2) % NQ)

    _load_idx(0, 0, 0)
    _load_idx(0, 1, 1)
    _wait_idx(0, 0, 0)
    _compute_ach(0)
    pltpu.async_copy(xw_hbm.at[gidx[0]], msg[0], semg[0])

    def _six(i, _):
        for r in range(NQ):
            _unit(NQ * i + r, r)
        return 0
    lax.fori_loop(0, n_units // NQ, _six, 0)

    _wait_scat((n_units - 2) % NB, (n_units - 2) % NQ)
    _wait_scat((n_units - 1) % NB, (n_units - 1) % NQ)

    plsc.subcore_barrier()
    pltpu.sync_copy(acc_sh.at[pl.ds(s * rows_per_tile, rows_per_tile)],
                    out_hbm.at[c, s])


def _sc_aggregate(src3, dst3, et3, atab, XW):
    n_nodes, d = XW.shape
    nchunk = src3.shape[1]
    rows_per_tile = n_nodes // NS
    zrows = 25
    mesh = plsc.VectorSubcoreMesh(core_axis_name="c", subcore_axis_name="s",
                                  num_cores=NC, num_subcores=NS)
    body = functools.partial(_sc_agg_body, n_nodes, nchunk, zrows)
    run = pl.kernel(
        body,
        out_type=jax.ShapeDtypeStruct((NC, NS, rows_per_tile, d),
                                      jnp.float32),
        mesh=mesh,
        scratch_types=[
            [pltpu.VMEM((CH,), jnp.int32)] * NQ,
            [pltpu.VMEM((CH,), jnp.int32)] * NQ,
            [pltpu.VMEM((CH,), jnp.int32)] * NQ,
            [pltpu.VMEM((CH,), jnp.float32)] * NQ,
            [pltpu.VMEM((CH, d), jnp.float32)] * NB,
            pltpu.VMEM((atab.shape[0],), jnp.float32),
            pltpu.VMEM_SHARED((n_nodes, d), jnp.float32),
            [pltpu.SemaphoreType.DMA] * NQ,
            [pltpu.SemaphoreType.DMA] * NB,
            [pltpu.SemaphoreType.DMA] * NB,
        ],
        compiler_params=pltpu.CompilerParams(needs_layout_passes=False),
    )
    return run(src3, dst3, et3, atab, XW)


def _fin_body(n_nodes, coef_ref, pr_ref, xw_ref, bias_ref, gam_ref, bet_ref,
              o_ref, sum_scr, sq_scr):
    p = pl.program_id(0)
    j = pl.program_id(1)
    t = (pr_ref[0] + pr_ref[1] + coef_ref[0, 0] * xw_ref[...]
         + bias_ref[...])

    @pl.when(p == 0)
    def _():
        @pl.when(j == 0)
        def _():
            sum_scr[...] = jnp.zeros_like(sum_scr)
            sq_scr[...] = jnp.zeros_like(sq_scr)
        sum_scr[...] += jnp.sum(t, axis=0, keepdims=True)
        sq_scr[...] += jnp.sum(t * t, axis=0, keepdims=True)

    @pl.when(p == 1)
    def _():
        mean = sum_scr[...] / n_nodes
        var = sq_scr[...] / n_nodes - mean * mean
        o_ref[...] = ((t - mean) * lax.rsqrt(var + EPS) * gam_ref[...]
                      + bet_ref[...])


def _finale(coef, pr, XW, bias, gamma, beta, blk):
    n, d = XW.shape
    nb = n // blk
    return pl.pallas_call(
        functools.partial(_fin_body, n),
        grid=(2, nb),
        in_specs=[
            pl.BlockSpec((1, 1), lambda p, j: (0, 0)),
            pl.BlockSpec((2, blk, d), lambda p, j: (0, j, 0)),
            pl.BlockSpec((blk, d), lambda p, j: (j, 0)),
            pl.BlockSpec((1, d), lambda p, j: (0, 0)),
            pl.BlockSpec((1, d), lambda p, j: (0, 0)),
            pl.BlockSpec((1, d), lambda p, j: (0, 0)),
        ],
        out_specs=pl.BlockSpec((blk, d), lambda p, j: (j, 0)),
        out_shape=jax.ShapeDtypeStruct((n, d), jnp.float32),
        scratch_shapes=[
            pltpu.VMEM((1, d), jnp.float32),
            pltpu.VMEM((1, d), jnp.float32),
        ],
    )(coef, pr, XW, bias, gamma, beta)


def kernel(x, r, edge_index, edge_type, W, bias, alpha, bn_gamma, bn_beta):
    n_nodes, d_in = x.shape
    d_out = W.shape[1]
    n_edges = edge_index.shape[1]
    n_alpha = alpha.shape[0]
    r_last = n_alpha - 2

    XW = _matmul(x.astype(jnp.float32), W.astype(jnp.float32), blk=1000)

    nchunk = 3 * (-(-n_edges // (NW * CH * 3)))
    epad = NW * nchunk * CH
    src = edge_index[0].astype(jnp.int32)
    dst = edge_index[1].astype(jnp.int32)
    et = edge_type.astype(jnp.int32)
    zpad = jnp.zeros((epad - n_edges,), jnp.int32)
    npad = (jnp.arange(epad - n_edges, dtype=jnp.int32) % n_nodes)
    src3 = jnp.concatenate([src, npad]).reshape(NW, nchunk, CH)
    dst3 = jnp.concatenate([dst, npad]).reshape(NW, nchunk, CH)
    et3 = jnp.concatenate([et, zpad]).reshape(NW, nchunk, CH)
    atab_len = -(-n_alpha // L) * L
    atab = jnp.pad(alpha[:, 0].astype(jnp.float32),
                   (0, atab_len - n_alpha))

    partial = _sc_aggregate(src3, dst3, et3, atab, XW)
    pr = partial.reshape(NC, n_nodes, d_out)

    coef = (2.0 * alpha[r_last]).astype(jnp.float32).reshape(1, 1)
    out = _finale(coef, pr, XW,
                  bias.astype(jnp.float32).reshape(1, d_out),
                  bn_gamma.astype(jnp.float32).reshape(1, d_out),
                  bn_beta.astype(jnp.float32).reshape(1, d_out),
                  blk=1000)
    return (out, r)


# speedup vs baseline: 23.1359x; 1.0369x over previous
import functools

import jax
import jax.numpy as jnp
from jax import lax
from jax.experimental import pallas as pl
from jax.experimental.pallas import tpu as pltpu
from jax.experimental.pallas import tpu_sc as plsc

EPS = 1e-5
NC = 2
NS = 16
NW = NC * NS
L = 16
CH = 112
NB = 3
NQ = 6


def _mm_body(x_ref, w_ref, o_ref):
    o_ref[...] = jnp.dot(x_ref[...], w_ref[...],
                         preferred_element_type=jnp.float32)


def _matmul(x, W, blk):
    n, d_in = x.shape
    d_out = W.shape[1]
    return pl.pallas_call(
        _mm_body,
        grid=(n // blk,),
        in_specs=[
            pl.BlockSpec((blk, d_in), lambda i: (i, 0)),
            pl.BlockSpec((d_in, d_out), lambda i: (0, 0)),
        ],
        out_specs=pl.BlockSpec((blk, d_out), lambda i: (i, 0)),
        out_shape=jax.ShapeDtypeStruct((n, d_out), jnp.float32),
    )(x, W)


def _sc_agg_body(n_nodes, nchunk, zrows,
                 src3_hbm, dst3_hbm, et3_hbm, alpha_hbm, xw_hbm, out_hbm,
                 gidx, sidx, etb, ach, msg, atab_v, acc_sh,
                 semi, semg, sems):
    c = lax.axis_index("c")
    s = lax.axis_index("s")
    w = s * NC + c
    rows_per_tile = n_nodes // NS
    n_units = 2 * nchunk

    pltpu.sync_copy(alpha_hbm, atab_v)

    def _zrow(i, _):
        for c8 in range(8):
            msg[0][i, pl.ds(c8 * L, L)] = jnp.zeros((L,), jnp.float32)
        return 0
    lax.fori_loop(0, zrows, _zrow, 0)
    for m in range(rows_per_tile // zrows):
        pltpu.sync_copy(
            msg[0].at[pl.ds(0, zrows)],
            acc_sh.at[pl.ds(s * rows_per_tile + m * zrows, zrows)])
    plsc.subcore_barrier()

    def _load_idx(u_j, u_d, q):
        g_hbm, s_hbm = (dst3_hbm, src3_hbm) if u_d == 0 else (src3_hbm,
                                                              dst3_hbm)
        pltpu.async_copy(g_hbm.at[w, u_j], gidx[q], semi[q])
        pltpu.async_copy(s_hbm.at[w, u_j], sidx[q], semi[q])
        pltpu.async_copy(et3_hbm.at[w, u_j], etb[q], semi[q])

    def _wait_idx(u_j, u_d, q):
        g_hbm, s_hbm = (dst3_hbm, src3_hbm) if u_d == 0 else (src3_hbm,
                                                              dst3_hbm)
        pltpu.make_async_copy(g_hbm.at[w, u_j], gidx[q], semi[q]).wait()
        pltpu.make_async_copy(s_hbm.at[w, u_j], sidx[q], semi[q]).wait()
        pltpu.make_async_copy(et3_hbm.at[w, u_j], etb[q], semi[q]).wait()

    def _compute_ach(q):
        for l in range(CH // L):
            et16 = etb[q][pl.ds(l * L, L)]
            ach[q][pl.ds(l * L, L)] = plsc.load_gather(atab_v, [et16])

    def _wait_scat(p, q):
        pltpu.make_async_copy(msg[p], acc_sh.at[sidx[q]], sems[p]).wait()

    def _unit(u, r):
        p, q = r % NB, r % NQ
        pn, qn = (r + 1) % NB, (r + 1) % NQ

        @pl.when(u + 1 < n_units)
        def _():
            nj, nd = (u + 1) // 2, (r + 1) % 2
            _wait_idx(nj, nd, qn)
            _compute_ach(qn)

            @pl.when(u >= 2)
            def _():
                _wait_scat(pn, (r + 4) % NQ)
            pltpu.async_copy(xw_hbm.at[gidx[qn]], msg[pn], semg[pn])

        pltpu.make_async_copy(xw_hbm.at[gidx[q]], msg[p], semg[p]).wait()

        @plsc.parallel_loop(0, CH, unroll=4)
        def _row(i):
            a = plsc.load_gather(ach[q], [jnp.full((L,), i, jnp.int32)])
            for c8 in range(8):
                sl = pl.ds(c8 * L, L)
                msg[p][i, sl] = msg[p][i, sl] * a

        pltpu.async_copy(msg[p], acc_sh.at[sidx[q]], sems[p], add=True)

        @pl.when(u + 2 < n_units)
        def _():
            nj, nd = (u + 2) // 2, r % 2
            _load_idx(nj, nd, (r + 2) % NQ)

    _load_idx(0, 0, 0)
    _load_idx(0, 1, 1)
    _wait_idx(0, 0, 0)
    _compute_ach(0)
    pltpu.async_copy(xw_hbm.at[gidx[0]], msg[0], semg[0])

    def _six(i, _):
        for r in range(NQ):
            _unit(NQ * i + r, r)
        return 0
    lax.fori_loop(0, n_units // NQ, _six, 0)

    _wait_scat((n_units - 2) % NB, (n_units - 2) % NQ)
    _wait_scat((n_units - 1) % NB, (n_units - 1) % NQ)

    plsc.subcore_barrier()
    pltpu.sync_copy(acc_sh.at[pl.ds(s * rows_per_tile, rows_per_tile)],
                    out_hbm.at[c, s])


def _sc_aggregate(src3, dst3, et3, atab, XW):
    n_nodes, d = XW.shape
    nchunk = src3.shape[1]
    rows_per_tile = n_nodes // NS
    zrows = 25
    mesh = plsc.VectorSubcoreMesh(core_axis_name="c", subcore_axis_name="s",
                                  num_cores=NC, num_subcores=NS)
    body = functools.partial(_sc_agg_body, n_nodes, nchunk, zrows)
    run = pl.kernel(
        body,
        out_type=jax.ShapeDtypeStruct((NC, NS, rows_per_tile, d),
                                      jnp.float32),
        mesh=mesh,
        scratch_types=[
            [pltpu.VMEM((CH,), jnp.int32)] * NQ,
            [pltpu.VMEM((CH,), jnp.int32)] * NQ,
            [pltpu.VMEM((CH,), jnp.int32)] * NQ,
            [pltpu.VMEM((CH,), jnp.float32)] * NQ,
            [pltpu.VMEM((CH, d), jnp.float32)] * NB,
            pltpu.VMEM((atab.shape[0],), jnp.float32),
            pltpu.VMEM_SHARED((n_nodes, d), jnp.float32),
            [pltpu.SemaphoreType.DMA] * NQ,
            [pltpu.SemaphoreType.DMA] * NB,
            [pltpu.SemaphoreType.DMA] * NB,
        ],
        compiler_params=pltpu.CompilerParams(needs_layout_passes=False),
    )
    return run(src3, dst3, et3, atab, XW)


def _fin_body(n_nodes, coef_ref, pr_ref, xw_ref, bias_ref, gam_ref, bet_ref,
              o_ref, sum_scr, sq_scr):
    p = pl.program_id(0)
    j = pl.program_id(1)
    t = (pr_ref[0] + pr_ref[1] + coef_ref[0, 0] * xw_ref[...]
         + bias_ref[...])

    @pl.when(p == 0)
    def _():
        @pl.when(j == 0)
        def _():
            sum_scr[...] = jnp.zeros_like(sum_scr)
            sq_scr[...] = jnp.zeros_like(sq_scr)
        sum_scr[...] += jnp.sum(t, axis=0, keepdims=True)
        sq_scr[...] += jnp.sum(t * t, axis=0, keepdims=True)

    @pl.when(p == 1)
    def _():
        mean = sum_scr[...] / n_nodes
        var = sq_scr[...] / n_nodes - mean * mean
        o_ref[...] = ((t - mean) * lax.rsqrt(var + EPS) * gam_ref[...]
                      + bet_ref[...])


def _finale(coef, pr, XW, bias, gamma, beta, blk):
    n, d = XW.shape
    nb = n // blk
    return pl.pallas_call(
        functools.partial(_fin_body, n),
        grid=(2, nb),
        in_specs=[
            pl.BlockSpec((1, 1), lambda p, j: (0, 0)),
            pl.BlockSpec((2, blk, d), lambda p, j: (0, j, 0)),
            pl.BlockSpec((blk, d), lambda p, j: (j, 0)),
            pl.BlockSpec((1, d), lambda p, j: (0, 0)),
            pl.BlockSpec((1, d), lambda p, j: (0, 0)),
            pl.BlockSpec((1, d), lambda p, j: (0, 0)),
        ],
        out_specs=pl.BlockSpec((blk, d), lambda p, j: (j, 0)),
        out_shape=jax.ShapeDtypeStruct((n, d), jnp.float32),
        scratch_shapes=[
            pltpu.VMEM((1, d), jnp.float32),
            pltpu.VMEM((1, d), jnp.float32),
        ],
    )(coef, pr, XW, bias, gamma, beta)


def kernel(x, r, edge_index, edge_type, W, bias, alpha, bn_gamma, bn_beta):
    n_nodes, d_in = x.shape
    d_out = W.shape[1]
    n_edges = edge_index.shape[1]
    n_alpha = alpha.shape[0]
    r_last = n_alpha - 2

    XW = _matmul(x.astype(jnp.float32), W.astype(jnp.float32), blk=1000)

    nchunk = 3 * (-(-n_edges // (NW * CH * 3)))
    epad = NW * nchunk * CH
    src = edge_index[0].astype(jnp.int32)
    dst = edge_index[1].astype(jnp.int32)
    et = edge_type.astype(jnp.int32)
    zpad = jnp.zeros((epad - n_edges,), jnp.int32)
    npad = (jnp.arange(epad - n_edges, dtype=jnp.int32) % n_nodes)
    src3 = jnp.concatenate([src, npad]).reshape(NW, nchunk, CH)
    dst3 = jnp.concatenate([dst, npad]).reshape(NW, nchunk, CH)
    et3 = jnp.concatenate([et, zpad]).reshape(NW, nchunk, CH)
    atab_len = -(-n_alpha // L) * L
    atab = jnp.pad(alpha[:, 0].astype(jnp.float32),
                   (0, atab_len - n_alpha))

    partial = _sc_aggregate(src3, dst3, et3, atab, XW)
    pr = partial.reshape(NC, n_nodes, d_out)

    coef = (2.0 * alpha[r_last]).astype(jnp.float32).reshape(1, 1)
    out = _finale(coef, pr, XW,
                  bias.astype(jnp.float32).reshape(1, d_out),
                  bn_gamma.astype(jnp.float32).reshape(1, d_out),
                  bn_beta.astype(jnp.float32).reshape(1, d_out),
                  blk=1000)
    return (out, r)


# speedup vs baseline: 25.9723x; 1.1226x over previous
import functools

import jax
import jax.numpy as jnp
from jax import lax
from jax.experimental import pallas as pl
from jax.experimental.pallas import tpu as pltpu
from jax.experimental.pallas import tpu_sc as plsc

EPS = 1e-5
NC = 2
NS = 16
NW = NC * NS
L = 16
CH = 112
NB = 2
NQ = 4


def _mm_body(x_ref, w_ref, o_ref):
    o_ref[...] = jnp.dot(x_ref[...], w_ref[...],
                         preferred_element_type=jnp.float32)


def _matmul(x, W, blk):
    n, d_in = x.shape
    d_out = W.shape[1]
    return pl.pallas_call(
        _mm_body,
        grid=(n // blk,),
        in_specs=[
            pl.BlockSpec((blk, d_in), lambda i: (i, 0)),
            pl.BlockSpec((d_in, d_out), lambda i: (0, 0)),
        ],
        out_specs=pl.BlockSpec((blk, d_out), lambda i: (i, 0)),
        out_shape=jax.ShapeDtypeStruct((n, d_out), jnp.float32),
    )(x, W)


def _sc_agg_body(n_nodes, nchunk, zrows,
                 src3_hbm, dst3_hbm, et3_hbm, alpha_hbm, xwh_hbm, out_hbm,
                 gidx, sidx, etb, ach, msgh, msg, atab_v, acc_sh,
                 semi, semg, sems):
    c = lax.axis_index("c")
    s = lax.axis_index("s")
    w = s * NC + c
    rows_per_tile = n_nodes // NS
    n_units = 2 * nchunk

    pltpu.sync_copy(alpha_hbm, atab_v)

    def _zrow(i, _):
        for c8 in range(8):
            msg[0][i, pl.ds(c8 * L, L)] = jnp.zeros((L,), jnp.float32)
        return 0
    lax.fori_loop(0, zrows, _zrow, 0)
    for m in range(rows_per_tile // zrows):
        pltpu.sync_copy(
            msg[0].at[pl.ds(0, zrows)],
            acc_sh.at[pl.ds(s * rows_per_tile + m * zrows, zrows)])
    plsc.subcore_barrier()

    def _load_idx(u_j, u_d, q):
        g_hbm, s_hbm = (dst3_hbm, src3_hbm) if u_d == 0 else (src3_hbm,
                                                              dst3_hbm)
        pltpu.async_copy(g_hbm.at[w, u_j], gidx[q], semi[q])
        pltpu.async_copy(s_hbm.at[w, u_j], sidx[q], semi[q])
        pltpu.async_copy(et3_hbm.at[w, u_j], etb[q], semi[q])

    def _wait_idx(u_j, u_d, q):
        g_hbm, s_hbm = (dst3_hbm, src3_hbm) if u_d == 0 else (src3_hbm,
                                                              dst3_hbm)
        pltpu.make_async_copy(g_hbm.at[w, u_j], gidx[q], semi[q]).wait()
        pltpu.make_async_copy(s_hbm.at[w, u_j], sidx[q], semi[q]).wait()
        pltpu.make_async_copy(et3_hbm.at[w, u_j], etb[q], semi[q]).wait()

    def _compute_ach(q):
        for l in range(CH // L):
            et16 = etb[q][pl.ds(l * L, L)]
            ach[q][pl.ds(l * L, L)] = plsc.load_gather(atab_v, [et16])

    def _wait_scat(p, q):
        pltpu.make_async_copy(msg[p], acc_sh.at[sidx[q]], sems[p]).wait()

    def _unit(u, r):
        p, q = r % NB, r % NQ
        pn, qn = (r + 1) % NB, (r + 1) % NQ

        @pl.when(u + 1 < n_units)
        def _():
            nj, nd = (u + 1) // 2, (r + 1) % 2
            _wait_idx(nj, nd, qn)
            _compute_ach(qn)
            pltpu.async_copy(xwh_hbm.at[gidx[qn]], msgh[pn], semg[pn])

        pltpu.make_async_copy(xwh_hbm.at[gidx[q]], msgh[p], semg[p]).wait()

        @pl.when(u >= 2)
        def _():
            _wait_scat(p, (r + 2) % NQ)

        @plsc.parallel_loop(0, CH, unroll=4)
        def _row(i):
            a = plsc.load_gather(ach[q], [jnp.full((L,), i, jnp.int32)])
            for c4 in range(4):
                mh = msgh[p][i, pl.ds(c4 * 2 * L, 2 * L)]
                lo, hi = plsc.unpack(mh,
                                     format=plsc.PackFormat.INTERLEAVED)
                msg[p][i, pl.ds(c4 * 2 * L, L)] = lo * a
                msg[p][i, pl.ds(c4 * 2 * L + L, L)] = hi * a

        pltpu.async_copy(msg[p], acc_sh.at[sidx[q]], sems[p], add=True)

        @pl.when(u + 2 < n_units)
        def _():
            nj, nd = (u + 2) // 2, r % 2
            _load_idx(nj, nd, (r + 2) % NQ)

    _load_idx(0, 0, 0)
    _load_idx(0, 1, 1)
    _wait_idx(0, 0, 0)
    _compute_ach(0)
    pltpu.async_copy(xwh_hbm.at[gidx[0]], msgh[0], semg[0])

    def _four(i, _):
        for r in range(NQ):
            _unit(NQ * i + r, r)
        return 0
    lax.fori_loop(0, n_units // NQ, _four, 0)

    _wait_scat((n_units - 2) % NB, (n_units - 2) % NQ)
    _wait_scat((n_units - 1) % NB, (n_units - 1) % NQ)

    plsc.subcore_barrier()
    pltpu.sync_copy(acc_sh.at[pl.ds(s * rows_per_tile, rows_per_tile)],
                    out_hbm.at[c, s])


def _sc_aggregate(src3, dst3, et3, atab, XWh):
    n_nodes, d = XWh.shape
    nchunk = src3.shape[1]
    rows_per_tile = n_nodes // NS
    zrows = 25
    mesh = plsc.VectorSubcoreMesh(core_axis_name="c", subcore_axis_name="s",
                                  num_cores=NC, num_subcores=NS)
    body = functools.partial(_sc_agg_body, n_nodes, nchunk, zrows)
    run = pl.kernel(
        body,
        out_type=jax.ShapeDtypeStruct((NC, NS, rows_per_tile, d),
                                      jnp.float32),
        mesh=mesh,
        scratch_types=[
            [pltpu.VMEM((CH,), jnp.int32)] * NQ,
            [pltpu.VMEM((CH,), jnp.int32)] * NQ,
            [pltpu.VMEM((CH,), jnp.int32)] * NQ,
            [pltpu.VMEM((CH,), jnp.float32)] * NQ,
            [pltpu.VMEM((CH, d), jnp.bfloat16)] * NB,
            [pltpu.VMEM((CH, d), jnp.float32)] * NB,
            pltpu.VMEM((atab.shape[0],), jnp.float32),
            pltpu.VMEM_SHARED((n_nodes, d), jnp.float32),
            [pltpu.SemaphoreType.DMA] * NQ,
            [pltpu.SemaphoreType.DMA] * NB,
            [pltpu.SemaphoreType.DMA] * NB,
        ],
        compiler_params=pltpu.CompilerParams(needs_layout_passes=False,
                                             use_tc_tiling_on_sc=False),
    )
    return run(src3, dst3, et3, atab, XWh)


def _fin_body(n_nodes, coef_ref, pr_ref, xw_ref, bias_ref, gam_ref, bet_ref,
              o_ref, sum_scr, sq_scr):
    p = pl.program_id(0)
    j = pl.program_id(1)
    t = (pr_ref[0] + pr_ref[1] + coef_ref[0, 0] * xw_ref[...]
         + bias_ref[...])

    @pl.when(p == 0)
    def _():
        @pl.when(j == 0)
        def _():
            sum_scr[...] = jnp.zeros_like(sum_scr)
            sq_scr[...] = jnp.zeros_like(sq_scr)
        sum_scr[...] += jnp.sum(t, axis=0, keepdims=True)
        sq_scr[...] += jnp.sum(t * t, axis=0, keepdims=True)

    @pl.when(p == 1)
    def _():
        mean = sum_scr[...] / n_nodes
        var = sq_scr[...] / n_nodes - mean * mean
        o_ref[...] = ((t - mean) * lax.rsqrt(var + EPS) * gam_ref[...]
                      + bet_ref[...])


def _finale(coef, pr, XW, bias, gamma, beta, blk):
    n, d = XW.shape
    nb = n // blk
    return pl.pallas_call(
        functools.partial(_fin_body, n),
        grid=(2, nb),
        in_specs=[
            pl.BlockSpec((1, 1), lambda p, j: (0, 0)),
            pl.BlockSpec((2, blk, d), lambda p, j: (0, j, 0)),
            pl.BlockSpec((blk, d), lambda p, j: (j, 0)),
            pl.BlockSpec((1, d), lambda p, j: (0, 0)),
            pl.BlockSpec((1, d), lambda p, j: (0, 0)),
            pl.BlockSpec((1, d), lambda p, j: (0, 0)),
        ],
        out_specs=pl.BlockSpec((blk, d), lambda p, j: (j, 0)),
        out_shape=jax.ShapeDtypeStruct((n, d), jnp.float32),
        scratch_shapes=[
            pltpu.VMEM((1, d), jnp.float32),
            pltpu.VMEM((1, d), jnp.float32),
        ],
    )(coef, pr, XW, bias, gamma, beta)


def kernel(x, r, edge_index, edge_type, W, bias, alpha, bn_gamma, bn_beta):
    n_nodes, d_in = x.shape
    d_out = W.shape[1]
    n_edges = edge_index.shape[1]
    n_alpha = alpha.shape[0]
    r_last = n_alpha - 2

    XW = _matmul(x.astype(jnp.float32), W.astype(jnp.float32), blk=1000)

    nchunk = 2 * (-(-n_edges // (NW * CH * 2)))
    epad = NW * nchunk * CH
    src = edge_index[0].astype(jnp.int32)
    dst = edge_index[1].astype(jnp.int32)
    et = edge_type.astype(jnp.int32)
    zpad = jnp.zeros((epad - n_edges,), jnp.int32)
    npad = (jnp.arange(epad - n_edges, dtype=jnp.int32) % n_nodes)
    src3 = jnp.concatenate([src, npad]).reshape(NW, nchunk, CH)
    dst3 = jnp.concatenate([dst, npad]).reshape(NW, nchunk, CH)
    et3 = jnp.concatenate([et, zpad]).reshape(NW, nchunk, CH)
    atab_len = -(-n_alpha // L) * L
    atab = jnp.pad(alpha[:, 0].astype(jnp.float32),
                   (0, atab_len - n_alpha))

    perm = [32 * c + o
            for c in range(d_out // 32)
            for k in range(L)
            for o in (k, L + k)]
    XWh = XW[:, jnp.array(perm, jnp.int32)].astype(jnp.bfloat16)

    partial = _sc_aggregate(src3, dst3, et3, atab, XWh)
    pr = partial.reshape(NC, n_nodes, d_out)

    coef = (2.0 * alpha[r_last]).astype(jnp.float32).reshape(1, 1)
    out = _finale(coef, pr, XW,
                  bias.astype(jnp.float32).reshape(1, d_out),
                  bn_gamma.astype(jnp.float32).reshape(1, d_out),
                  bn_beta.astype(jnp.float32).reshape(1, d_out),
                  blk=1000)
    return (out, r)
